# baseline (speedup 1.0000x reference)
import functools

import jax
import jax.numpy as jnp
from jax import lax
from jax.experimental import pallas as pl
from jax.experimental.pallas import tpu as pltpu
from jax.experimental.pallas import tpu_sc as plsc

V = 1_000_000
D = 32
S = 26
B = 16384
CHUNK = 128
NCHUNKS_TOT = (S * B) // CHUNK
NC = 2
NS = 16
NW = NC * NS
CPW = NCHUNKS_TOT // NW
NBUF = 2

TBLK = 512
TGRID = -(-V // TBLK)

_mesh = plsc.VectorSubcoreMesh(core_axis_name="c", subcore_axis_name="s")


def _repack_body(w_ref, o_ref, m_ref):
    m_ref[...] = w_ref[...].T
    for k in range(4):
        o_ref[:, k * D:(k + 1) * D] = m_ref[pl.Slice(k, TBLK // 4, 4), :]


_repack = pl.pallas_call(
    _repack_body,
    grid=(TGRID,),
    in_specs=[pl.BlockSpec((D, TBLK), lambda b: (0, b))],
    out_specs=pl.BlockSpec((TBLK // 4, 4 * D), lambda b: (b, 0)),
    out_shape=jax.ShapeDtypeStruct((V // 4, 4 * D), jnp.float32),
    scratch_shapes=[pltpu.VMEM((TBLK, D), jnp.float32)],
)


@functools.partial(
    pl.kernel,
    mesh=_mesh,
    out_type=jax.ShapeDtypeStruct((S, D // 8, B // CHUNK, 8, CHUNK),
                                  jnp.float32),
    scratch_types=[
        pltpu.VMEM((CPW, CHUNK), jnp.int32),
        pltpu.VMEM((NBUF, CHUNK, D), jnp.float32),
        pltpu.VMEM((NBUF, D // 8, 1, 8, CHUNK), jnp.float32),
        [pltpu.SemaphoreType.DMA] * NBUF,
        [pltpu.SemaphoreType.DMA] * NBUF,
    ],
    compiler_params=pltpu.CompilerParams(use_tc_tiling_on_sc=False,
                                         needs_layout_passes=False,
                                         disable_bounds_checks=True),
)
def _gather_kernel(idx_hbm, table_hbm, out_hbm, idx_v, bufg, buft, sg, sw):
    wid = lax.axis_index("s") * NC + lax.axis_index("c")
    c0 = wid * CPW
    pltpu.sync_copy(idx_hbm.at[pl.ds(c0, CPW)], idx_v)

    def gather(j, b):
        return pltpu.make_async_copy(
            table_hbm.at[idx_v.at[j]], bufg.at[b], sg[b])

    def write(j, b):
        cg = c0 + j
        s = cg // (B // CHUNK)
        bh = cg % (B // CHUNK)
        return pltpu.make_async_copy(
            buft.at[b], out_hbm.at[s, :, pl.ds(bh, 1)], sw[b])

    def transpose(b):
        src = bufg.at[b]

        @plsc.parallel_loop(0, (CHUNK // 16) * D, 1, unroll=8)
        def _(t):
            v = t % (CHUNK // 16)
            c = t // (CHUNK // 16)
            rows = jnp.arange(16, dtype=jnp.int32) + v * 16
            cols = jnp.zeros((16,), dtype=jnp.int32) + c
            vreg = plsc.load_gather(src, [rows, cols])
            buft[b, c // 8, 0, c % 8, pl.ds(v * 16, 16)] = vreg

    for b in range(NBUF):
        gather(b, b).start()
    for b in range(NBUF):
        gather(b, b).wait()
        transpose(b)
        write(b, b).start()
        gather(b + NBUF, b).start()

    def group(g, carry):
        for b in range(NBUF):
            j = g * NBUF + b
            gather(j, b).wait()
            write(j - NBUF, b).wait()
            transpose(b)
            write(j, b).start()
            gather(j + NBUF, b).start()
        return carry

    lax.fori_loop(1, CPW // NBUF - 1, group, 0)

    for b in range(NBUF):
        j = CPW - NBUF + b
        gather(j, b).wait()
        write(j - NBUF, b).wait()
        transpose(b)
        write(j, b).start()
    for b in range(NBUF):
        write(CPW - NBUF + b, b).wait()


def kernel(token_ids, weight):
    table = _repack(weight.astype(jnp.float32).T).reshape(V, D)
    ids = token_ids.astype(jnp.int32).T.reshape(NCHUNKS_TOT, CHUNK)
    out5 = _gather_kernel(ids, table)
    return out5.transpose(2, 4, 0, 1, 3).reshape(B, S, D)


# speedup vs baseline: 2.1092x; 2.1092x over previous
import functools

import jax
import jax.numpy as jnp
from jax import lax
from jax.experimental import pallas as pl
from jax.experimental.pallas import tpu as pltpu
from jax.experimental.pallas import tpu_sc as plsc

V = 1_000_000
D = 32
S = 26
B = 16384
CHUNK = 128
NCHUNKS_TOT = (S * B) // CHUNK
NC = 2
NS = 16
NW = NC * NS
CPW = NCHUNKS_TOT // NW
NBUF = 2

TBLK = 2048
TGRID = -(-V // TBLK)

_mesh = plsc.VectorSubcoreMesh(core_axis_name="c", subcore_axis_name="s")


def _repack_body(w_ref, o_ref, m_ref):
    for sb in range(TBLK // 512):
        m_ref[pl.ds(sb * 512, 512), :] = w_ref[:, pl.ds(sb * 512, 512)].T
    for k in range(4):
        o_ref[:, k * D:(k + 1) * D] = m_ref[pl.Slice(k, TBLK // 4, 4), :]


_repack = pl.pallas_call(
    _repack_body,
    grid=(TGRID,),
    in_specs=[pl.BlockSpec((D, TBLK), lambda b: (0, b))],
    out_specs=pl.BlockSpec((TBLK // 4, 4 * D), lambda b: (b, 0)),
    out_shape=jax.ShapeDtypeStruct((V // 4, 4 * D), jnp.float32),
    scratch_shapes=[pltpu.VMEM((TBLK, D), jnp.float32)],
)


@functools.partial(
    pl.kernel,
    mesh=_mesh,
    out_type=jax.ShapeDtypeStruct((S, D // 8, B // CHUNK, 8, CHUNK),
                                  jnp.float32),
    scratch_types=[
        pltpu.VMEM((CPW, CHUNK), jnp.int32),
        pltpu.VMEM((NBUF, CHUNK, D), jnp.float32),
        pltpu.VMEM((NBUF, D // 8, 1, 8, CHUNK), jnp.float32),
        [pltpu.SemaphoreType.DMA] * NBUF,
        [pltpu.SemaphoreType.DMA] * NBUF,
    ],
    compiler_params=pltpu.CompilerParams(use_tc_tiling_on_sc=False,
                                         needs_layout_passes=False,
                                         disable_bounds_checks=True),
)
def _gather_kernel(idx_hbm, table_hbm, out_hbm, idx_v, bufg, buft, sg, sw):
    wid = lax.axis_index("s") * NC + lax.axis_index("c")
    c0 = wid * CPW
    pltpu.sync_copy(idx_hbm.at[pl.ds(c0, CPW)], idx_v)

    def gather(j, b):
        return pltpu.make_async_copy(
            table_hbm.at[idx_v.at[j]], bufg.at[b], sg[b])

    def write(j, b):
        cg = c0 + j
        s = cg // (B // CHUNK)
        bh = cg % (B // CHUNK)
        return pltpu.make_async_copy(
            buft.at[b], out_hbm.at[s, :, pl.ds(bh, 1)], sw[b])

    def transpose(b):
        src = bufg.at[b]

        @plsc.parallel_loop(0, (CHUNK // 16) * D, 1, unroll=8)
        def _(t):
            v = t % (CHUNK // 16)
            c = t // (CHUNK // 16)
            rows = jnp.arange(16, dtype=jnp.int32) + v * 16
            cols = jnp.zeros((16,), dtype=jnp.int32) + c
            vreg = plsc.load_gather(src, [rows, cols])
            buft[b, c // 8, 0, c % 8, pl.ds(v * 16, 16)] = vreg

    for b in range(NBUF):
        gather(b, b).start()
    for b in range(NBUF):
        gather(b, b).wait()
        transpose(b)
        write(b, b).start()
        gather(b + NBUF, b).start()

    def group(g, carry):
        for b in range(NBUF):
            j = g * NBUF + b
            gather(j, b).wait()
            write(j - NBUF, b).wait()
            transpose(b)
            write(j, b).start()
            gather(j + NBUF, b).start()
        return carry

    lax.fori_loop(1, CPW // NBUF - 1, group, 0)

    for b in range(NBUF):
        j = CPW - NBUF + b
        gather(j, b).wait()
        write(j - NBUF, b).wait()
        transpose(b)
        write(j, b).start()
    for b in range(NBUF):
        write(CPW - NBUF + b, b).wait()


def kernel(token_ids, weight):
    table = _repack(weight.astype(jnp.float32).T).reshape(V, D)
    ids = token_ids.astype(jnp.int32).T.reshape(NCHUNKS_TOT, CHUNK)
    out5 = _gather_kernel(ids, table)
    return out5.transpose(2, 4, 0, 1, 3).reshape(B, S, D)


# speedup vs baseline: 2.6727x; 1.2672x over previous
import functools

import jax
import jax.numpy as jnp
from jax import lax
from jax.experimental import pallas as pl
from jax.experimental.pallas import tpu as pltpu
from jax.experimental.pallas import tpu_sc as plsc

V = 1_000_000
D = 32
S = 26
B = 16384
CHUNK = 128
NCHUNKS_TOT = (S * B) // CHUNK
NC = 2
NS = 16
NW = NC * NS
CPW = NCHUNKS_TOT // NW
NBUF = 2

TBLK = 8192
TGRID = -(-V // TBLK)

_mesh = plsc.VectorSubcoreMesh(core_axis_name="c", subcore_axis_name="s")


def _repack_body(w_ref, o_ref, m_ref):
    for sb in range(TBLK // 512):
        m_ref[pl.ds(sb * 512, 512), :] = w_ref[:, pl.ds(sb * 512, 512)].T
    for k in range(4):
        o_ref[:, k * D:(k + 1) * D] = m_ref[pl.Slice(k, TBLK // 4, 4), :]


_repack = pl.pallas_call(
    _repack_body,
    grid=(TGRID,),
    in_specs=[pl.BlockSpec((D, TBLK), lambda b: (0, b))],
    out_specs=pl.BlockSpec((TBLK // 4, 4 * D), lambda b: (b, 0)),
    out_shape=jax.ShapeDtypeStruct((V // 4, 4 * D), jnp.float32),
    scratch_shapes=[pltpu.VMEM((TBLK, D), jnp.float32)],
)


@functools.partial(
    pl.kernel,
    mesh=_mesh,
    out_type=jax.ShapeDtypeStruct((S, D // 8, B // CHUNK, 8, CHUNK),
                                  jnp.float32),
    scratch_types=[
        pltpu.VMEM((CPW, CHUNK), jnp.int32),
        pltpu.VMEM((NBUF, CHUNK, D), jnp.float32),
        pltpu.VMEM((NBUF, D // 8, 1, 8, CHUNK), jnp.float32),
        [pltpu.SemaphoreType.DMA] * NBUF,
        [pltpu.SemaphoreType.DMA] * NBUF,
    ],
    compiler_params=pltpu.CompilerParams(use_tc_tiling_on_sc=False,
                                         needs_layout_passes=False,
                                         disable_bounds_checks=True),
)
def _gather_kernel(idx_hbm, table_hbm, out_hbm, idx_v, bufg, buft, sg, sw):
    wid = lax.axis_index("s") * NC + lax.axis_index("c")
    c0 = wid * CPW
    pltpu.sync_copy(idx_hbm.at[pl.ds(c0, CPW)], idx_v)

    def gather(j, b):
        return pltpu.make_async_copy(
            table_hbm.at[idx_v.at[j]], bufg.at[b], sg[b])

    def write(j, b):
        cg = c0 + j
        s = cg // (B // CHUNK)
        bh = cg % (B // CHUNK)
        return pltpu.make_async_copy(
            buft.at[b], out_hbm.at[s, :, pl.ds(bh, 1)], sw[b])

    def transpose(b):
        src = bufg.at[b]

        @plsc.parallel_loop(0, (CHUNK // 16) * D, 1, unroll=8)
        def _(t):
            v = t % (CHUNK // 16)
            c = t // (CHUNK // 16)
            rows = jnp.arange(16, dtype=jnp.int32) + v * 16
            cols = jnp.zeros((16,), dtype=jnp.int32) + c
            vreg = plsc.load_gather(src, [rows, cols])
            buft[b, c // 8, 0, c % 8, pl.ds(v * 16, 16)] = vreg

    for b in range(NBUF):
        gather(b, b).start()
    for b in range(NBUF):
        gather(b, b).wait()
        transpose(b)
        write(b, b).start()
        gather(b + NBUF, b).start()

    def group(g, carry):
        for b in range(NBUF):
            j = g * NBUF + b
            gather(j, b).wait()
            write(j - NBUF, b).wait()
            transpose(b)
            write(j, b).start()
            gather(j + NBUF, b).start()
        return carry

    lax.fori_loop(1, CPW // NBUF - 1, group, 0)

    for b in range(NBUF):
        j = CPW - NBUF + b
        gather(j, b).wait()
        write(j - NBUF, b).wait()
        transpose(b)
        write(j, b).start()
    for b in range(NBUF):
        write(CPW - NBUF + b, b).wait()


def kernel(token_ids, weight):
    table = _repack(weight.astype(jnp.float32).T).reshape(V, D)
    ids = token_ids.astype(jnp.int32).T.reshape(NCHUNKS_TOT, CHUNK)
    out5 = _gather_kernel(ids, table)
    return out5.transpose(2, 4, 0, 1, 3).reshape(B, S, D)


# speedup vs baseline: 2.7075x; 1.0130x over previous
import functools

import jax
import jax.numpy as jnp
from jax import lax
from jax.experimental import pallas as pl
from jax.experimental.pallas import tpu as pltpu
from jax.experimental.pallas import tpu_sc as plsc

V = 1_000_000
D = 32
S = 26
B = 16384
CHUNK = 128
NCHUNKS_TOT = (S * B) // CHUNK
NC = 2
NS = 16
NW = NC * NS
CPW = NCHUNKS_TOT // NW
NBUF = 2

TBLK = 8192
TGRID = -(-V // TBLK)

_mesh = plsc.VectorSubcoreMesh(core_axis_name="c", subcore_axis_name="s")


def _repack_body(w_ref, o_ref, m_ref):
    for sb in range(TBLK // 512):
        m_ref[sb] = w_ref[:, pl.ds(sb * 512, 512)].T
        for k in range(4):
            o_ref[pl.ds(sb * 128, 128), k * D:(k + 1) * D] = (
                m_ref[sb, pl.Slice(k, 128, 4), :])


_repack = pl.pallas_call(
    _repack_body,
    grid=(TGRID,),
    in_specs=[pl.BlockSpec((D, TBLK), lambda b: (0, b))],
    out_specs=pl.BlockSpec((TBLK // 4, 4 * D), lambda b: (b, 0)),
    out_shape=jax.ShapeDtypeStruct((V // 4, 4 * D), jnp.float32),
    scratch_shapes=[pltpu.VMEM((TBLK // 512, 512, D), jnp.float32)],
)


@functools.partial(
    pl.kernel,
    mesh=_mesh,
    out_type=jax.ShapeDtypeStruct((S, D // 8, B // CHUNK, 8, CHUNK),
                                  jnp.float32),
    scratch_types=[
        pltpu.VMEM((CPW, CHUNK), jnp.int32),
        pltpu.VMEM((NBUF, CHUNK, D), jnp.float32),
        pltpu.VMEM((NBUF, D // 8, 1, 8, CHUNK), jnp.float32),
        [pltpu.SemaphoreType.DMA] * NBUF,
        [pltpu.SemaphoreType.DMA] * NBUF,
    ],
    compiler_params=pltpu.CompilerParams(use_tc_tiling_on_sc=False,
                                         needs_layout_passes=False,
                                         disable_bounds_checks=True),
)
def _gather_kernel(idx_hbm, table_hbm, out_hbm, idx_v, bufg, buft, sg, sw):
    wid = lax.axis_index("s") * NC + lax.axis_index("c")
    c0 = wid * CPW
    pltpu.sync_copy(idx_hbm.at[pl.ds(c0, CPW)], idx_v)

    def gather(j, b):
        return pltpu.make_async_copy(
            table_hbm.at[idx_v.at[j]], bufg.at[b], sg[b])

    def write(j, b):
        cg = c0 + j
        s = cg // (B // CHUNK)
        bh = cg % (B // CHUNK)
        return pltpu.make_async_copy(
            buft.at[b], out_hbm.at[s, :, pl.ds(bh, 1)], sw[b])

    def transpose(b):
        src = bufg.at[b]

        @plsc.parallel_loop(0, (CHUNK // 16) * D, 1, unroll=16)
        def _(t):
            v = t % (CHUNK // 16)
            c = t // (CHUNK // 16)
            rows = jnp.arange(16, dtype=jnp.int32) + v * 16
            cols = jnp.zeros((16,), dtype=jnp.int32) + c
            vreg = plsc.load_gather(src, [rows, cols])
            buft[b, c // 8, 0, c % 8, pl.ds(v * 16, 16)] = vreg

    for b in range(NBUF):
        gather(b, b).start()
    for b in range(NBUF):
        gather(b, b).wait()
        transpose(b)
        write(b, b).start()
        gather(b + NBUF, b).start()

    def group(g, carry):
        for b in range(NBUF):
            j = g * NBUF + b
            gather(j, b).wait()
            write(j - NBUF, b).wait()
            transpose(b)
            write(j, b).start()
            gather(j + NBUF, b).start()
        return carry

    lax.fori_loop(1, CPW // NBUF - 1, group, 0)

    for b in range(NBUF):
        j = CPW - NBUF + b
        gather(j, b).wait()
        write(j - NBUF, b).wait()
        transpose(b)
        write(j, b).start()
    for b in range(NBUF):
        write(CPW - NBUF + b, b).wait()


def kernel(token_ids, weight):
    table = _repack(weight.astype(jnp.float32).T).reshape(V, D)
    ids = token_ids.astype(jnp.int32).T.reshape(NCHUNKS_TOT, CHUNK)
    out5 = _gather_kernel(ids, table)
    return out5.transpose(2, 4, 0, 1, 3).reshape(B, S, D)


# speedup vs baseline: 2.7727x; 1.0241x over previous
import functools

import jax
import jax.numpy as jnp
from jax import lax
from jax.experimental import pallas as pl
from jax.experimental.pallas import tpu as pltpu
from jax.experimental.pallas import tpu_sc as plsc

V = 1_000_000
D = 32
S = 26
B = 16384
CHUNK = 128
NCHUNKS_TOT = (S * B) // CHUNK
NC = 2
NS = 16
NW = NC * NS
CPW = NCHUNKS_TOT // NW
NBUF = 2

TBLK = 16384
TGRID = -(-V // TBLK)

_mesh = plsc.VectorSubcoreMesh(core_axis_name="c", subcore_axis_name="s")


def _repack_body(w_ref, o_ref, m_ref):
    for sb in range(TBLK // 512):
        m_ref[sb] = w_ref[:, pl.ds(sb * 512, 512)].T
        for k in range(4):
            o_ref[pl.ds(sb * 128, 128), k * D:(k + 1) * D] = (
                m_ref[sb, pl.Slice(k, 128, 4), :])


_repack = pl.pallas_call(
    _repack_body,
    grid=(TGRID,),
    in_specs=[pl.BlockSpec((D, TBLK), lambda b: (0, b))],
    out_specs=pl.BlockSpec((TBLK // 4, 4 * D), lambda b: (b, 0)),
    out_shape=jax.ShapeDtypeStruct((V // 4, 4 * D), jnp.float32),
    scratch_shapes=[pltpu.VMEM((TBLK // 512, 512, D), jnp.float32)],
)


@functools.partial(
    pl.kernel,
    mesh=_mesh,
    out_type=jax.ShapeDtypeStruct((S, D // 8, B // CHUNK, 8, CHUNK),
                                  jnp.float32),
    scratch_types=[
        pltpu.VMEM((CPW, CHUNK), jnp.int32),
        pltpu.VMEM((NBUF, CHUNK, D), jnp.float32),
        pltpu.VMEM((NBUF, D // 8, 1, 8, CHUNK), jnp.float32),
        [pltpu.SemaphoreType.DMA] * NBUF,
        [pltpu.SemaphoreType.DMA] * NBUF,
    ],
    compiler_params=pltpu.CompilerParams(use_tc_tiling_on_sc=False,
                                         needs_layout_passes=False,
                                         disable_bounds_checks=True),
)
def _gather_kernel(idx_hbm, table_hbm, out_hbm, idx_v, bufg, buft, sg, sw):
    wid = lax.axis_index("s") * NC + lax.axis_index("c")
    c0 = wid * CPW
    pltpu.sync_copy(idx_hbm.at[pl.ds(c0, CPW)], idx_v)

    def gather(j, b):
        return pltpu.make_async_copy(
            table_hbm.at[idx_v.at[j]], bufg.at[b], sg[b])

    def write(j, b):
        cg = c0 + j
        s = cg // (B // CHUNK)
        bh = cg % (B // CHUNK)
        return pltpu.make_async_copy(
            buft.at[b], out_hbm.at[s, :, pl.ds(bh, 1)], sw[b])

    def transpose(b):
        src = bufg.at[b]

        @plsc.parallel_loop(0, (CHUNK // 16) * D, 1, unroll=32)
        def _(t):
            v = t % (CHUNK // 16)
            c = t // (CHUNK // 16)
            rows = jnp.arange(16, dtype=jnp.int32) + v * 16
            cols = jnp.zeros((16,), dtype=jnp.int32) + c
            vreg = plsc.load_gather(src, [rows, cols])
            buft[b, c // 8, 0, c % 8, pl.ds(v * 16, 16)] = vreg

    for b in range(NBUF):
        gather(b, b).start()
    for b in range(NBUF):
        gather(b, b).wait()
        transpose(b)
        write(b, b).start()
        gather(b + NBUF, b).start()

    def group(g, carry):
        for b in range(NBUF):
            j = g * NBUF + b
            gather(j, b).wait()
            write(j - NBUF, b).wait()
            transpose(b)
            write(j, b).start()
            gather(j + NBUF, b).start()
        return carry

    lax.fori_loop(1, CPW // NBUF - 1, group, 0)

    for b in range(NBUF):
        j = CPW - NBUF + b
        gather(j, b).wait()
        write(j - NBUF, b).wait()
        transpose(b)
        write(j, b).start()
    for b in range(NBUF):
        write(CPW - NBUF + b, b).wait()


def kernel(token_ids, weight):
    table = _repack(weight.astype(jnp.float32).T).reshape(V, D)
    ids = token_ids.astype(jnp.int32).T.reshape(NCHUNKS_TOT, CHUNK)
    out5 = _gather_kernel(ids, table)
    return out5.transpose(2, 4, 0, 1, 3).reshape(B, S, D)
